# baseline (speedup 1.0000x reference)
import functools

import jax
import jax.numpy as jnp
from jax import lax
from jax.experimental import pallas as pl
from jax.experimental.pallas import tpu as pltpu
from jax.experimental.pallas import tpu_sc as plsc
import numpy as np

B = 8
N = 4096
C = 128
S = 1024
NS = 32
RADIUS2 = np.float32(0.2 ** 2)
ROW = 144

SC_CORES = 2
SC_SUBCORES = 16
NW = SC_CORES * SC_SUBCORES
QPW = (B * S) // NW
QCHUNK = 4
NCHUNK = QPW // QCHUNK


def _fps_body(xyzT_ref, fps_ref, nxT_ref, d2_ref):
    xv = xyzT_ref[0]
    yv = xyzT_ref[1]
    zv = xyzT_ref[2]
    d2_ref[...] = xv * xv + yv * yv + zv * zv
    lane = lax.broadcasted_iota(jnp.int32, (B, N), 1)
    lane128 = lax.broadcasted_iota(jnp.int32, (B, 128), 1)

    def body(i, carry):
        dist, far, bi, bx, by, bz = carry
        bi = jnp.where(lane128 == 127, far, pltpu.roll(bi, 127, 1))
        oh = lane == far
        cx = jnp.sum(jnp.where(oh, xv, 0.0), axis=1, keepdims=True)
        cy = jnp.sum(jnp.where(oh, yv, 0.0), axis=1, keepdims=True)
        cz = jnp.sum(jnp.where(oh, zv, 0.0), axis=1, keepdims=True)
        bx = jnp.where(lane128 == 127, cx, pltpu.roll(bx, 127, 1))
        by = jnp.where(lane128 == 127, cy, pltpu.roll(by, 127, 1))
        bz = jnp.where(lane128 == 127, cz, pltpu.roll(bz, 127, 1))
        dx = xv - cx
        dy = yv - cy
        dz = zv - cz
        d = dx * dx + dy * dy + dz * dz
        dist = jnp.minimum(dist, d)
        m = jnp.max(dist, axis=1, keepdims=True)
        far = jnp.min(jnp.where(dist == m, lane, N), axis=1, keepdims=True)
        return dist, far, bi, bx, by, bz

    dist = jnp.full((B, N), 1e10, jnp.float32)
    far = jnp.zeros((B, 1), jnp.int32)
    zi = jnp.zeros((B, 128), jnp.int32)
    zf = jnp.zeros((B, 128), jnp.float32)
    for g in range(S // 128):
        dist, far, bi, bx, by, bz = lax.fori_loop(
            0, 128, body, (dist, far, zi, zf, zf, zf))
        fps_ref[:, g * 128:(g + 1) * 128] = bi
        nxT_ref[0, :, g * 128:(g + 1) * 128] = bx
        nxT_ref[1, :, g * 128:(g + 1) * 128] = by
        nxT_ref[2, :, g * 128:(g + 1) * 128] = bz


def _fps(xyzT, interpret=False):
    return pl.pallas_call(
        _fps_body,
        out_shape=(
            jax.ShapeDtypeStruct((B, S), jnp.int32),
            jax.ShapeDtypeStruct((3, B, S), jnp.float32),
            jax.ShapeDtypeStruct((B, N), jnp.float32),
        ),
        interpret=interpret,
    )(xyzT)


NW16 = N // 16


def _mm(a, b_):
    return lax.dot_general(a, b_, (((1,), (0,)), ((), ())),
                           preferred_element_type=jnp.float32)


def _mask_body(nx_ref, xyzT_ref, d2_ref, pmat_ref, p2_ref, t256_ref,
               out_ref, cprev_ref, nzpre_ref, hdr_ref):
    b = pl.program_id(0)
    nx = nx_ref[0]
    x3 = xyzT_ref[:, b, :]
    s2 = (nx[:, 0:1] * nx[:, 0:1] + nx[:, 1:2] * nx[:, 1:2]) \
        + nx[:, 2:3] * nx[:, 2:3]
    dot = _mm(nx.astype(jnp.bfloat16), x3.astype(jnp.bfloat16))
    sq = (s2 + d2_ref[b][None, :]) - 2.0 * dot
    m01f = jnp.where(sq > RADIUS2, 0.0, 1.0)
    m01 = m01f.astype(jnp.bfloat16)
    packed = _mm(m01, pmat_ref[...])
    cprev = _mm(m01, p2_ref[...])
    active = jnp.where((packed > 0.0) & (cprev < float(NS)), 1.0, 0.0)
    nzpre = _mm(active.astype(jnp.bfloat16), t256_ref[...])
    ctot = jnp.sum(m01f, axis=1, keepdims=True)
    nztot = jnp.sum(active, axis=1, keepdims=True)
    out_ref[0] = packed.astype(jnp.int32)
    cprev_ref[0] = cprev.astype(jnp.int32)
    nzpre_ref[0] = nzpre.astype(jnp.int32)
    hdr_ref[0] = jnp.concatenate([ctot, nztot], axis=1).astype(jnp.int32)


def _mask(new_xyz, xyzT, d2, pmat, p2, t256):
    return pl.pallas_call(
        _mask_body,
        grid=(B, S // 128),
        in_specs=[
            pl.BlockSpec((1, 128, 3), lambda b, q: (b, q, 0)),
            pl.BlockSpec((3, B, N), lambda b, q: (0, 0, 0)),
            pl.BlockSpec((B, N), lambda b, q: (0, 0)),
            pl.BlockSpec((N, NW16), lambda b, q: (0, 0)),
            pl.BlockSpec((N, NW16), lambda b, q: (0, 0)),
            pl.BlockSpec((NW16, NW16), lambda b, q: (0, 0)),
        ],
        out_specs=(
            pl.BlockSpec((1, 128, NW16), lambda b, q: (b, q, 0)),
            pl.BlockSpec((1, 128, NW16), lambda b, q: (b, q, 0)),
            pl.BlockSpec((1, 128, NW16), lambda b, q: (b, q, 0)),
            pl.BlockSpec((1, 128, 2), lambda b, q: (b, q, 0)),
        ),
        out_shape=(
            jax.ShapeDtypeStruct((B, S, NW16), jnp.int32),
            jax.ShapeDtypeStruct((B, S, NW16), jnp.int32),
            jax.ShapeDtypeStruct((B, S, NW16), jnp.int32),
            jax.ShapeDtypeStruct((B, S, 2), jnp.int32),
        ),
    )(new_xyz, xyzT, d2, pmat, p2, t256)


def _grouper_body(words_hbm, cprev_hbm, nzpre_hbm, hdr_hbm,
                  nx_hbm, table_hbm, out_hbm,
                  nxv, nzbuf, selbuf,
                  wbuf0, wbuf1, cbuf0, cbuf1, pbuf0, pbuf1, hbuf0, hbuf1,
                  gidx0, gidx1, rows0, rows1,
                  otile0, otile1, gsem0, gsem1):
    wid = lax.axis_index("s") * SC_CORES + lax.axis_index("c")
    b = wid // 4
    s0 = (wid % 4) * QPW

    pltpu.sync_copy(nx_hbm.at[b, pl.ds(s0 * 3, QPW * 3)], nxv)

    iota = lax.broadcasted_iota(jnp.int32, (16,), 0)
    zero16 = jnp.zeros((16,), jnp.int32)
    bufs = ((wbuf0, cbuf0, pbuf0, hbuf0, gidx0, rows0, otile0, gsem0),
            (wbuf1, cbuf1, pbuf1, hbuf1, gidx1, rows1, otile1, gsem1))

    def select_fire(qc, wbuf, cbuf, pbuf, hbuf, gidx, rows, otile, gsem):
        del otile
        w0 = (s0 + qc * QCHUNK) * NW16
        pltpu.sync_copy(words_hbm.at[b, pl.ds(w0, QCHUNK * NW16)], wbuf)
        pltpu.sync_copy(cprev_hbm.at[b, pl.ds(w0, QCHUNK * NW16)], cbuf)
        pltpu.sync_copy(nzpre_hbm.at[b, pl.ds(w0, QCHUNK * NW16)], pbuf)
        pltpu.sync_copy(hdr_hbm.at[b, pl.ds((s0 + qc * QCHUNK) * 2,
                                            QCHUNK * 2)], hbuf)

        def do_query(j, _):
            base = j * NW16
            ctot = plsc.load_gather(hbuf, [zero16 + 2 * j])
            nzw = plsc.load_gather(hbuf, [zero16 + (2 * j + 1)])

            def _l1(t, _):
                wv = wbuf[pl.ds(base + t * 16, 16)]
                cp = cbuf[pl.ds(base + t * 16, 16)]
                rk = pbuf[pl.ds(base + t * 16, 16)]
                act = jnp.logical_and(wv != 0, cp < NS)
                plsc.store_scatter(nzbuf, [rk], t * 16 + iota, mask=act)
                return 0

            lax.fori_loop(0, NW16 // 16, _l1, 0)

            def _l2(i, _):
                valid = (zero16 + i) < nzw
                widx = plsc.load_gather(nzbuf, [zero16 + i], mask=valid)
                wsplat = plsc.load_gather(wbuf, [base + widx], mask=valid)
                offs = plsc.load_gather(cbuf, [base + widx], mask=valid)
                mask = jnp.logical_and(((wsplat >> iota) & 1) != 0, valid)
                pos = offs + plsc.cumsum(jnp.where(mask, 1, 0)) - 1
                plsc.store_scatter(selbuf, [pos], widx * 16 + iota,
                                   mask=mask)
                return 0

            lax.fori_loop(0, NS, _l2, 0)

            cntv = jnp.minimum(ctot, NS)
            chunk0 = selbuf[pl.ds(0, 16)]
            fmask = iota < jnp.minimum(cntv, 16)
            first = zero16 + jnp.min(jnp.where(fmask, chunk0, N))
            for h in range(2):
                cur = selbuf[pl.ds(h * 16, 16)]
                kio = iota + h * 16
                sel = jnp.where(kio < cntv, cur, first)
                gidx[pl.ds(j * NS + h * 16, 16)] = sel + b * N
            return 0

        lax.fori_loop(0, QCHUNK, do_query, 0)
        pltpu.async_copy(table_hbm.at[gidx], rows, gsem)

    def transpose(qc, rows, otile):
        def per_query(q, _):
            sv = plsc.load_gather(nxv, [3 * (qc * QCHUNK + q)
                                        + jnp.minimum(iota, 2)])
            sv = jnp.where(iota < 3, sv, 0.0)
            qv = zero16 + q

            def _pk(k_, _):
                p = q * NS + k_
                kv = zero16 + k_
                for h in range(9):
                    vals = rows[p, pl.ds(h * 16, 16)]
                    civ = iota + h * 16
                    if h == 0:
                        vals = vals - sv
                    if h == 8:
                        plsc.store_scatter(otile, [civ, qv, kv], vals,
                                           mask=civ < (C + 3))
                    else:
                        plsc.store_scatter(otile, [civ, qv, kv], vals)
                return 0

            lax.fori_loop(0, NS, _pk, 0)
            return 0

        lax.fori_loop(0, QCHUNK, per_query, 0)

    def drain_transpose_out(qc, wbuf, cbuf, pbuf, hbuf, gidx, rows, otile,
                            gsem, fire_next):
        pltpu.make_async_copy(table_hbm.at[gidx], rows, gsem).wait()
        transpose(qc, rows, otile)
        if fire_next:
            select_fire(qc + 2, wbuf, cbuf, pbuf, hbuf, gidx, rows, None,
                        gsem)
        pltpu.sync_copy(otile,
                        out_hbm.at[b, :, pl.ds(s0 + qc * QCHUNK, QCHUNK), :])

    select_fire(0, *bufs[0])
    select_fire(1, *bufs[1])

    def pair(h, _):
        for par in range(2):
            drain_transpose_out(2 * h + par, *bufs[par], fire_next=True)
        return 0

    lax.fori_loop(0, NCHUNK // 2 - 1, pair, 0)
    for par in range(2):
        drain_transpose_out(NCHUNK - 2 + par, *bufs[par], fire_next=False)


@functools.lru_cache(maxsize=None)
def _make_grouper():
    @functools.partial(
        pl.kernel,
        out_type=jax.ShapeDtypeStruct((B, C + 3, S, NS), jnp.float32),
        mesh=plsc.VectorSubcoreMesh(core_axis_name="c", subcore_axis_name="s"),
        compiler_params=pltpu.CompilerParams(needs_layout_passes=False,
                                             use_tc_tiling_on_sc=False),
        scratch_types=[
            pltpu.VMEM((QPW * 3,), jnp.float32),
            pltpu.VMEM((NS,), jnp.int32),
            pltpu.VMEM((48,), jnp.int32),
            pltpu.VMEM((QCHUNK * NW16,), jnp.int32),
            pltpu.VMEM((QCHUNK * NW16,), jnp.int32),
            pltpu.VMEM((QCHUNK * NW16,), jnp.int32),
            pltpu.VMEM((QCHUNK * NW16,), jnp.int32),
            pltpu.VMEM((QCHUNK * NW16,), jnp.int32),
            pltpu.VMEM((QCHUNK * NW16,), jnp.int32),
            pltpu.VMEM((QCHUNK * 2,), jnp.int32),
            pltpu.VMEM((QCHUNK * 2,), jnp.int32),
            pltpu.VMEM((QCHUNK * NS,), jnp.int32),
            pltpu.VMEM((QCHUNK * NS,), jnp.int32),
            pltpu.VMEM((QCHUNK * NS, ROW), jnp.float32),
            pltpu.VMEM((QCHUNK * NS, ROW), jnp.float32),
            pltpu.VMEM((C + 3, QCHUNK, NS), jnp.float32),
            pltpu.VMEM((C + 3, QCHUNK, NS), jnp.float32),
            pltpu.SemaphoreType.DMA,
            pltpu.SemaphoreType.DMA,
        ],
    )
    def _grouper(words_hbm, cprev_hbm, nzpre_hbm, hdr_hbm, nx_hbm, table_hbm,
                 out_hbm, *scratch):
        _grouper_body(words_hbm, cprev_hbm, nzpre_hbm, hdr_hbm, nx_hbm,
                      table_hbm, out_hbm, *scratch)

    return _grouper


def _consts():
    n = jnp.arange(N)[:, None]
    w = jnp.arange(NW16)[None, :]
    pmat = jnp.where(n // 16 == w, 2.0 ** (n % 16), 0.0).astype(jnp.bfloat16)
    p2 = jnp.where(n < 16 * w, 1.0, 0.0).astype(jnp.bfloat16)
    wr = jnp.arange(NW16)[:, None]
    t256 = jnp.where(wr < w, 1.0, 0.0).astype(jnp.bfloat16)
    return pmat, p2, t256


def kernel(xyz, features):
    xyzT = jnp.transpose(xyz, (2, 0, 1))
    fps_idx, nxT, d2 = _fps(xyzT)
    new_xyz = jnp.transpose(nxT, (1, 2, 0))
    pmat, p2, t256 = _consts()
    words, cprev, nzpre, hdr = _mask(new_xyz, xyzT, d2, pmat, p2, t256)
    featT = jnp.transpose(features, (0, 2, 1))
    pad = jnp.zeros((B, N, ROW - C - 3), jnp.float32)
    table = jnp.concatenate([xyz, featT, pad], axis=2).reshape(B * N, ROW)
    nx_flat = new_xyz.reshape(B, S * 3)
    new_features = _make_grouper()(
        words.reshape(B, S * NW16), cprev.reshape(B, S * NW16),
        nzpre.reshape(B, S * NW16), hdr.reshape(B, S * 2), nx_flat, table)
    return (new_xyz, new_features, fps_idx)


# speedup vs baseline: 1.3985x; 1.3985x over previous
import functools

import jax
import jax.numpy as jnp
from jax import lax
from jax.experimental import pallas as pl
from jax.experimental.pallas import tpu as pltpu
from jax.experimental.pallas import tpu_sc as plsc
import numpy as np

B = 8
N = 4096
C = 128
S = 1024
NS = 32
RADIUS2 = np.float32(0.2 ** 2)
ROW = 144

SC_CORES = 2
SC_SUBCORES = 16
NW = SC_CORES * SC_SUBCORES
QPW = (B * S) // NW
QCHUNK = 4
NCHUNK = QPW // QCHUNK


def _fps_body(xyzT_ref, fps_ref, nxT_ref, d2_ref):
    xv = xyzT_ref[0]
    yv = xyzT_ref[1]
    zv = xyzT_ref[2]
    d2_ref[...] = xv * xv + yv * yv + zv * zv
    lane = lax.broadcasted_iota(jnp.int32, (B, N), 1)
    lane128 = lax.broadcasted_iota(jnp.int32, (B, 128), 1)

    def body(i, carry):
        dist, far, bi, bx, by, bz = carry
        bi = jnp.where(lane128 == 127, far, pltpu.roll(bi, 127, 1))
        oh = lane == far
        cx = jnp.sum(jnp.where(oh, xv, 0.0), axis=1, keepdims=True)
        cy = jnp.sum(jnp.where(oh, yv, 0.0), axis=1, keepdims=True)
        cz = jnp.sum(jnp.where(oh, zv, 0.0), axis=1, keepdims=True)
        bx = jnp.where(lane128 == 127, cx, pltpu.roll(bx, 127, 1))
        by = jnp.where(lane128 == 127, cy, pltpu.roll(by, 127, 1))
        bz = jnp.where(lane128 == 127, cz, pltpu.roll(bz, 127, 1))
        dx = xv - cx
        dy = yv - cy
        dz = zv - cz
        d = dx * dx + dy * dy + dz * dz
        dist = jnp.minimum(dist, d)
        m = jnp.max(dist, axis=1, keepdims=True)
        far = jnp.min(jnp.where(dist == m, lane, N), axis=1, keepdims=True)
        return dist, far, bi, bx, by, bz

    dist = jnp.full((B, N), 1e10, jnp.float32)
    far = jnp.zeros((B, 1), jnp.int32)
    zi = jnp.zeros((B, 128), jnp.int32)
    zf = jnp.zeros((B, 128), jnp.float32)
    for g in range(S // 128):
        dist, far, bi, bx, by, bz = lax.fori_loop(
            0, 128, body, (dist, far, zi, zf, zf, zf))
        fps_ref[:, g * 128:(g + 1) * 128] = bi
        nxT_ref[0, :, g * 128:(g + 1) * 128] = bx
        nxT_ref[1, :, g * 128:(g + 1) * 128] = by
        nxT_ref[2, :, g * 128:(g + 1) * 128] = bz


def _fps(xyzT, interpret=False):
    return pl.pallas_call(
        _fps_body,
        out_shape=(
            jax.ShapeDtypeStruct((B, S), jnp.int32),
            jax.ShapeDtypeStruct((3, B, S), jnp.float32),
            jax.ShapeDtypeStruct((B, N), jnp.float32),
        ),
        interpret=interpret,
    )(xyzT)


NW16 = N // 16


def _mm(a, b_):
    return lax.dot_general(a, b_, (((1,), (0,)), ((), ())),
                           preferred_element_type=jnp.float32)


def _mask_body(nx_ref, xyzT_ref, d2_ref, pmat_ref, p2_ref, t256_ref,
               out_ref, cprev_ref, nzpre_ref, hdr_ref):
    b = pl.program_id(0)
    nx = nx_ref[0]
    x3 = xyzT_ref[:, b, :]
    s2 = (nx[:, 0:1] * nx[:, 0:1] + nx[:, 1:2] * nx[:, 1:2]) \
        + nx[:, 2:3] * nx[:, 2:3]
    dot = _mm(nx.astype(jnp.bfloat16), x3.astype(jnp.bfloat16))
    sq = (s2 + d2_ref[b][None, :]) - 2.0 * dot
    m01f = jnp.where(sq > RADIUS2, 0.0, 1.0)
    m01 = m01f.astype(jnp.bfloat16)
    packed = _mm(m01, pmat_ref[...])
    cprev = _mm(m01, p2_ref[...])
    active = jnp.where((packed > 0.0) & (cprev < float(NS)), 1.0, 0.0)
    nzpre = _mm(active.astype(jnp.bfloat16), t256_ref[...])
    ctot = jnp.sum(m01f, axis=1, keepdims=True)
    nztot = jnp.sum(active, axis=1, keepdims=True)
    out_ref[0] = packed.astype(jnp.int32)
    cprev_ref[0] = cprev.astype(jnp.int32)
    nzpre_ref[0] = nzpre.astype(jnp.int32)
    hdr_ref[0] = jnp.concatenate([ctot, nztot], axis=1).astype(jnp.int32)


def _mask(new_xyz, xyzT, d2, pmat, p2, t256):
    return pl.pallas_call(
        _mask_body,
        grid=(B, S // 128),
        in_specs=[
            pl.BlockSpec((1, 128, 3), lambda b, q: (b, q, 0)),
            pl.BlockSpec((3, B, N), lambda b, q: (0, 0, 0)),
            pl.BlockSpec((B, N), lambda b, q: (0, 0)),
            pl.BlockSpec((N, NW16), lambda b, q: (0, 0)),
            pl.BlockSpec((N, NW16), lambda b, q: (0, 0)),
            pl.BlockSpec((NW16, NW16), lambda b, q: (0, 0)),
        ],
        out_specs=(
            pl.BlockSpec((1, 128, NW16), lambda b, q: (b, q, 0)),
            pl.BlockSpec((1, 128, NW16), lambda b, q: (b, q, 0)),
            pl.BlockSpec((1, 128, NW16), lambda b, q: (b, q, 0)),
            pl.BlockSpec((1, 128, 2), lambda b, q: (b, q, 0)),
        ),
        out_shape=(
            jax.ShapeDtypeStruct((B, S, NW16), jnp.int32),
            jax.ShapeDtypeStruct((B, S, NW16), jnp.int32),
            jax.ShapeDtypeStruct((B, S, NW16), jnp.int32),
            jax.ShapeDtypeStruct((B, S, 2), jnp.int32),
        ),
    )(new_xyz, xyzT, d2, pmat, p2, t256)


def _grouper_body(words_hbm, cprev_hbm, nzpre_hbm, hdr_hbm,
                  nx_hbm, table_hbm, out_hbm,
                  nxv, nzbuf, selbuf,
                  wbuf0, wbuf1, cbuf0, cbuf1, pbuf0, pbuf1, hbuf0, hbuf1,
                  gidx0, gidx1, rows0, rows1,
                  otile0, otile1, gsem0, gsem1):
    wid = lax.axis_index("s") * SC_CORES + lax.axis_index("c")
    b = wid // 4
    s0 = (wid % 4) * QPW

    pltpu.sync_copy(nx_hbm.at[b, pl.ds(s0 * 3, QPW * 3)], nxv)

    iota = lax.broadcasted_iota(jnp.int32, (16,), 0)
    zero16 = jnp.zeros((16,), jnp.int32)
    bufs = ((wbuf0, cbuf0, pbuf0, hbuf0, gidx0, rows0, otile0, gsem0),
            (wbuf1, cbuf1, pbuf1, hbuf1, gidx1, rows1, otile1, gsem1))

    def select_fire(qc, wbuf, cbuf, pbuf, hbuf, gidx, rows, otile, gsem):
        del otile
        w0 = (s0 + qc * QCHUNK) * NW16
        pltpu.sync_copy(words_hbm.at[b, pl.ds(w0, QCHUNK * NW16)], wbuf)
        pltpu.sync_copy(cprev_hbm.at[b, pl.ds(w0, QCHUNK * NW16)], cbuf)
        pltpu.sync_copy(nzpre_hbm.at[b, pl.ds(w0, QCHUNK * NW16)], pbuf)
        pltpu.sync_copy(hdr_hbm.at[b, pl.ds((s0 + qc * QCHUNK) * 2,
                                            QCHUNK * 2)], hbuf)

        def do_query(j, _):
            base = j * NW16
            ctot = plsc.load_gather(hbuf, [zero16 + 2 * j])
            nzw = plsc.load_gather(hbuf, [zero16 + (2 * j + 1)])

            def _l1(t, _):
                wv = wbuf[pl.ds(base + t * 16, 16)]
                cp = cbuf[pl.ds(base + t * 16, 16)]
                rk = pbuf[pl.ds(base + t * 16, 16)]
                act = jnp.logical_and(wv != 0, cp < NS)
                plsc.store_scatter(nzbuf, [rk], t * 16 + iota, mask=act)
                return 0

            lax.fori_loop(0, NW16 // 16, _l1, 0)

            def _l2(i, _):
                valid = (zero16 + i) < nzw
                widx = plsc.load_gather(nzbuf, [zero16 + i], mask=valid)
                wsplat = plsc.load_gather(wbuf, [base + widx], mask=valid)
                offs = plsc.load_gather(cbuf, [base + widx], mask=valid)
                mask = jnp.logical_and(((wsplat >> iota) & 1) != 0, valid)
                pos = offs + plsc.cumsum(jnp.where(mask, 1, 0)) - 1
                plsc.store_scatter(selbuf, [pos], widx * 16 + iota,
                                   mask=mask)
                return 0

            lax.fori_loop(0, NS, _l2, 0)

            cntv = jnp.minimum(ctot, NS)
            chunk0 = selbuf[pl.ds(0, 16)]
            fmask = iota < jnp.minimum(cntv, 16)
            first = zero16 + jnp.min(jnp.where(fmask, chunk0, N))
            for h in range(2):
                cur = selbuf[pl.ds(h * 16, 16)]
                kio = iota + h * 16
                sel = jnp.where(kio < cntv, cur, first)
                gidx[pl.ds(j * NS + h * 16, 16)] = sel + b * N
            return 0

        lax.fori_loop(0, QCHUNK, do_query, 0)
        pltpu.async_copy(table_hbm.at[gidx], rows, gsem)

    def transpose(qc, rows, otile):
        def per_query(q, _):
            sv = plsc.load_gather(nxv, [3 * (qc * QCHUNK + q)
                                        + jnp.minimum(iota, 2)])
            sv = jnp.where(iota < 3, sv, 0.0)
            qv = zero16 + q

            @functools.partial(plsc.parallel_loop, 0, NS, unroll=2)
            def _pk(k_):
                p = q * NS + k_
                kv = zero16 + k_
                for h in range(9):
                    vals = rows[p, pl.ds(h * 16, 16)]
                    civ = iota + h * 16
                    if h == 0:
                        vals = vals - sv
                    if h == 8:
                        plsc.store_scatter(otile, [civ, qv, kv], vals,
                                           mask=civ < (C + 3))
                    else:
                        plsc.store_scatter(otile, [civ, qv, kv], vals)

            return 0

        lax.fori_loop(0, QCHUNK, per_query, 0)

    def drain_transpose_out(qc, wbuf, cbuf, pbuf, hbuf, gidx, rows, otile,
                            gsem, fire_next):
        pltpu.make_async_copy(table_hbm.at[gidx], rows, gsem).wait()
        transpose(qc, rows, otile)
        if fire_next:
            select_fire(qc + 2, wbuf, cbuf, pbuf, hbuf, gidx, rows, None,
                        gsem)
        pltpu.sync_copy(otile,
                        out_hbm.at[b, :, pl.ds(s0 + qc * QCHUNK, QCHUNK), :])

    select_fire(0, *bufs[0])
    select_fire(1, *bufs[1])

    def pair(h, _):
        for par in range(2):
            drain_transpose_out(2 * h + par, *bufs[par], fire_next=True)
        return 0

    lax.fori_loop(0, NCHUNK // 2 - 1, pair, 0)
    for par in range(2):
        drain_transpose_out(NCHUNK - 2 + par, *bufs[par], fire_next=False)


@functools.lru_cache(maxsize=None)
def _make_grouper():
    @functools.partial(
        pl.kernel,
        out_type=jax.ShapeDtypeStruct((B, C + 3, S, NS), jnp.float32),
        mesh=plsc.VectorSubcoreMesh(core_axis_name="c", subcore_axis_name="s"),
        compiler_params=pltpu.CompilerParams(needs_layout_passes=False,
                                             use_tc_tiling_on_sc=False),
        scratch_types=[
            pltpu.VMEM((QPW * 3,), jnp.float32),
            pltpu.VMEM((NS,), jnp.int32),
            pltpu.VMEM((48,), jnp.int32),
            pltpu.VMEM((QCHUNK * NW16,), jnp.int32),
            pltpu.VMEM((QCHUNK * NW16,), jnp.int32),
            pltpu.VMEM((QCHUNK * NW16,), jnp.int32),
            pltpu.VMEM((QCHUNK * NW16,), jnp.int32),
            pltpu.VMEM((QCHUNK * NW16,), jnp.int32),
            pltpu.VMEM((QCHUNK * NW16,), jnp.int32),
            pltpu.VMEM((QCHUNK * 2,), jnp.int32),
            pltpu.VMEM((QCHUNK * 2,), jnp.int32),
            pltpu.VMEM((QCHUNK * NS,), jnp.int32),
            pltpu.VMEM((QCHUNK * NS,), jnp.int32),
            pltpu.VMEM((QCHUNK * NS, ROW), jnp.float32),
            pltpu.VMEM((QCHUNK * NS, ROW), jnp.float32),
            pltpu.VMEM((C + 3, QCHUNK, NS), jnp.float32),
            pltpu.VMEM((C + 3, QCHUNK, NS), jnp.float32),
            pltpu.SemaphoreType.DMA,
            pltpu.SemaphoreType.DMA,
        ],
    )
    def _grouper(words_hbm, cprev_hbm, nzpre_hbm, hdr_hbm, nx_hbm, table_hbm,
                 out_hbm, *scratch):
        _grouper_body(words_hbm, cprev_hbm, nzpre_hbm, hdr_hbm, nx_hbm,
                      table_hbm, out_hbm, *scratch)

    return _grouper


def _consts():
    n = jnp.arange(N)[:, None]
    w = jnp.arange(NW16)[None, :]
    pmat = jnp.where(n // 16 == w, 2.0 ** (n % 16), 0.0).astype(jnp.bfloat16)
    p2 = jnp.where(n < 16 * w, 1.0, 0.0).astype(jnp.bfloat16)
    wr = jnp.arange(NW16)[:, None]
    t256 = jnp.where(wr < w, 1.0, 0.0).astype(jnp.bfloat16)
    return pmat, p2, t256


def kernel(xyz, features):
    xyzT = jnp.transpose(xyz, (2, 0, 1))
    fps_idx, nxT, d2 = _fps(xyzT)
    new_xyz = jnp.transpose(nxT, (1, 2, 0))
    pmat, p2, t256 = _consts()
    words, cprev, nzpre, hdr = _mask(new_xyz, xyzT, d2, pmat, p2, t256)
    featT = jnp.transpose(features, (0, 2, 1))
    pad = jnp.zeros((B, N, ROW - C - 3), jnp.float32)
    table = jnp.concatenate([xyz, featT, pad], axis=2).reshape(B * N, ROW)
    nx_flat = new_xyz.reshape(B, S * 3)
    new_features = _make_grouper()(
        words.reshape(B, S * NW16), cprev.reshape(B, S * NW16),
        nzpre.reshape(B, S * NW16), hdr.reshape(B, S * 2), nx_flat, table)
    return (new_xyz, new_features, fps_idx)
